# baseline (speedup 1.0000x reference)
import jax
import jax.numpy as jnp
from jax.experimental import pallas as pl
from jax.experimental.pallas import tpu as pltpu

B = 256
HW = 32
E = 16
K = 2
CAP = 128
NUM_CLASSES = 10
TCDIM = 256

T = 8


def _conv3x3(xp, w, T_, H, W, Cin, Cout, splits=None):
    rows = T_ * H * W
    xs = [xp[:, dy:dy + H, dx:dx + W, :].reshape(rows, Cin)
          for dy in range(3) for dx in range(3)]
    lhs = jnp.concatenate(xs, axis=1)
    wf = w.reshape(9 * Cin, Cout)
    if splits is None:
        return jnp.dot(lhs, wf, preferred_element_type=jnp.float32)
    acc = jnp.zeros((rows, Cout), jnp.float32)
    k0 = 0
    for kl in splits:
        acc = acc + jnp.dot(lhs[:, k0:k0 + kl], wf[k0:k0 + kl],
                            preferred_element_type=jnp.float32)
        k0 += kl
    return acc


def _pool2x2(x, T_, H, W, C):
    xh = x.reshape(T_, H // 2, 2, W, C)
    xh = jnp.maximum(xh[:, :, 0], xh[:, :, 1])
    cols = []
    for w2 in range(W // 2):
        m = jnp.maximum(xh[:, :, 2 * w2, :], xh[:, :, 2 * w2 + 1, :])
        cols.append(m[:, :, None, :])
    return jnp.concatenate(cols, axis=2)


def _bnrelu(z, s):
    return jnp.maximum(((z + s[0:1]) - s[1:2]) / s[2:3] * s[3:4] + s[4:5],
                       0.0)


def _trunk_body(xp_ref, w1_ref, s1_ref, w2_ref, s2_ref, w3_ref, s3_ref,
                w4_ref, s4_ref, out_ref, p1_ref, p3_ref, p4_ref):
    @pl.when(pl.program_id(0) == 0)
    def _():
        p1_ref[...] = jnp.zeros_like(p1_ref)
        p3_ref[...] = jnp.zeros_like(p3_ref)
        p4_ref[...] = jnp.zeros_like(p4_ref)

    xp = xp_ref[...].reshape(T, 34, 34, 3)
    a1 = _conv3x3(xp, w1_ref[...], T, 32, 32, 3, 32)
    a1 = _bnrelu(a1, s1_ref)
    p1_ref[:, 1:33, 1:33, :] = a1.reshape(T, 32, 32, 32)
    a2 = _conv3x3(p1_ref[...], w2_ref[...], T, 32, 32, 32, 64)
    a2 = _bnrelu(a2, s2_ref)
    a2 = _pool2x2(a2.reshape(T, 32, 32, 64), T, 32, 32, 64)
    p3_ref[:, 1:17, 1:17, :] = a2
    a3 = _conv3x3(p3_ref[...], w3_ref[...], T, 16, 16, 64, 128)
    a3 = _bnrelu(a3, s3_ref)
    p4_ref[:, 1:17, 1:17, :] = a3.reshape(T, 16, 16, 128)
    a4 = _conv3x3(p4_ref[...], w4_ref[...], T, 16, 16, 128, 256)
    a4 = _bnrelu(a4, s4_ref)
    a4 = _pool2x2(a4.reshape(T, 16, 16, 256), T, 16, 16, 256)
    out_ref[...] = jnp.mean(a4.reshape(T, 64, 256), axis=1)[None]


def _moe_body(f_ref, gw1_ref, gb1_ref, gw2_ref, gb2_ref, cw_ref, cb_ref,
              fl_ref, sc_ref, dm_ref):
    f = f_ref[...]
    h = jnp.maximum(jnp.dot(f, gw1_ref[...],
                            preferred_element_type=jnp.float32)
                    + gb1_ref[...], 0.0)
    gl = jnp.dot(h, gw2_ref[...],
                 preferred_element_type=jnp.float32) + gb2_ref[...]
    logits = []
    score_cols = []
    for e in range(E):
        le = jnp.dot(f, cw_ref[e],
                     preferred_element_type=jnp.float32) + cb_ref[e]
        m = jnp.max(le, axis=1, keepdims=True)
        p = jnp.exp(le - m)
        probs = p / jnp.sum(p, axis=1, keepdims=True)
        ent = -jnp.sum(probs * jnp.log(jnp.clip(probs, 1e-12, None)),
                       axis=1, keepdims=True)
        logits.append(le)
        score_cols.append(gl[:, e:e + 1] * (-ent))
    sc = jnp.concatenate(score_cols, axis=1)
    sc_ref[...] = sc
    scT = sc.T
    ii = jax.lax.broadcasted_iota(jnp.int32, (B, B), 0)
    jj = jax.lax.broadcasted_iota(jnp.int32, (B, B), 1)
    d1_cols = []
    for e in range(E):
        si = sc[:, e:e + 1]
        sj = scT[e:e + 1, :]
        ahead = (sj > si) | ((sj == si) & (jj < ii))
        rank = jnp.sum(ahead.astype(jnp.float32), axis=1, keepdims=True)
        d1_cols.append((rank < float(min(CAP, B))).astype(jnp.float32))
    d1 = jnp.concatenate(d1_cols, axis=1)
    masked = jnp.where(d1 > 0.5, sc, -1e9)
    lane = jax.lax.broadcasted_iota(jnp.int32, (B, E), 1)
    d_cols = []
    for e in range(E):
        me = masked[:, e:e + 1]
        ahead = (masked > me) | ((masked == me) & (lane < e))
        rank = jnp.sum(ahead.astype(jnp.float32), axis=1, keepdims=True)
        d_cols.append((rank < float(K)).astype(jnp.float32))
    df = jnp.concatenate(d_cols, axis=1)
    dm_ref[...] = df
    w = sc * df
    acc = jnp.zeros((B, NUM_CLASSES), jnp.float32)
    for e in range(E):
        acc = acc + w[:, e:e + 1] * logits[e]
    norm = jnp.clip(jnp.sum(df, axis=1, keepdims=True), 1.0, None)
    fl_ref[...] = acc / norm


def _trunk_call(x, params, bn_stats):
    f32 = jnp.float32
    xp = jnp.pad(jnp.transpose(x, (0, 2, 3, 1)),
                 ((0, 0), (1, 1), (1, 1), (0, 0)))
    xm = xp.reshape(B, 34, 34 * 3)

    def stats(cb, g, bb, m, v):
        return jnp.stack([cb, m, jnp.sqrt(v + 1e-5), g, bb], axis=0)

    w1 = jnp.transpose(params['c1w'], (2, 3, 1, 0))
    w2 = jnp.transpose(params['c2w'], (2, 3, 1, 0))
    w3 = jnp.transpose(params['c3w'], (2, 3, 1, 0))
    w4 = jnp.transpose(params['c4w'], (2, 3, 1, 0))
    s1 = stats(params['c1b'], params['g1'], params['b1'],
               bn_stats['m1'], bn_stats['v1'])
    s2 = stats(params['c2b'], params['g2'], params['b2'],
               bn_stats['m2'], bn_stats['v2'])
    s3 = stats(params['c3b'], params['g3'], params['b3'],
               bn_stats['m3'], bn_stats['v3'])
    s4 = stats(params['c4b'], params['g4'], params['b4'],
               bn_stats['m4'], bn_stats['v4'])

    full = lambda shape: pl.BlockSpec(shape, lambda *a: (0,) * len(shape))
    feats = pl.pallas_call(
        _trunk_body,
        grid=(B // T,),
        in_specs=[
            pl.BlockSpec((T, 34, 102), lambda i: (i, 0, 0)),
            full((3, 3, 3, 32)), full((5, 32)),
            full((3, 3, 32, 64)), full((5, 64)),
            full((3, 3, 64, 128)), full((5, 128)),
            full((3, 3, 128, 256)), full((5, 256)),
        ],
        out_specs=pl.BlockSpec((1, T, TCDIM), lambda i: (i, 0, 0)),
        out_shape=jax.ShapeDtypeStruct((B // T, T, TCDIM), f32),
        scratch_shapes=[
            pltpu.VMEM((T, 34, 34, 32), f32),
            pltpu.VMEM((T, 18, 18, 64), f32),
            pltpu.VMEM((T, 18, 18, 128), f32),
        ],
    )(xm, w1, s1, w2, s2, w3, s3, w4, s4)
    return feats.reshape(B, TCDIM)


def _moe_call(feats, params):
    f32 = jnp.float32
    full = lambda shape: pl.BlockSpec(shape, lambda *a: (0,) * len(shape))
    gw1T = params['gw1'].T
    gb1 = params['gb1'][None, :]
    gw2T = params['gw2'].T
    gb2 = params['gb2'][None, :]
    cwT = jnp.transpose(params['cls_w'], (0, 2, 1))
    cb = params['cls_b'][:, None, :]

    fl, sc, dm = pl.pallas_call(
        _moe_body,
        in_specs=[
            full((B, TCDIM)),
            full((TCDIM, TCDIM // 2)), full((1, TCDIM // 2)),
            full((TCDIM // 2, E)), full((1, E)),
            full((E, TCDIM, NUM_CLASSES)), full((E, 1, NUM_CLASSES)),
        ],
        out_specs=[full((B, NUM_CLASSES)), full((B, E)), full((B, E))],
        out_shape=[
            jax.ShapeDtypeStruct((B, NUM_CLASSES), f32),
            jax.ShapeDtypeStruct((B, E), f32),
            jax.ShapeDtypeStruct((B, E), f32),
        ],
    )(feats, gw1T, gb1, gw2T, gb2, cwT, cb)
    return (fl, sc, dm)


def kernel(x, params, bn_stats):
    feats = _trunk_call(x, params, bn_stats)
    fl, sc, dm = _moe_call(feats, params)
    return (fl, sc, dm > 0.5)


# speedup vs baseline: 1.0010x; 1.0010x over previous
import jax
import jax.numpy as jnp
from jax.experimental import pallas as pl
from jax.experimental.pallas import tpu as pltpu

B = 256
HW = 32
E = 16
K = 2
CAP = 128
NUM_CLASSES = 10
TCDIM = 256

T = 8


def _conv3x3(xp, w, T_, H, W, Cin, Cout, splits=None):
    rows = T_ * H * W
    xs = [xp[:, dy:dy + H, dx:dx + W, :].reshape(rows, Cin)
          for dy in range(3) for dx in range(3)]
    lhs = jnp.concatenate(xs, axis=1)
    wf = w.reshape(9 * Cin, Cout)
    if splits is None:
        return jnp.dot(lhs, wf, preferred_element_type=jnp.float32)
    acc = jnp.zeros((rows, Cout), jnp.float32)
    k0 = 0
    for kl in splits:
        acc = acc + jnp.dot(lhs[:, k0:k0 + kl], wf[k0:k0 + kl],
                            preferred_element_type=jnp.float32)
        k0 += kl
    return acc


def _pool2x2(x, T_, H, W, C):
    xh = x.reshape(T_, H // 2, 2, W, C)
    xh = jnp.maximum(xh[:, :, 0], xh[:, :, 1])
    cols = []
    for w2 in range(W // 2):
        m = jnp.maximum(xh[:, :, 2 * w2, :], xh[:, :, 2 * w2 + 1, :])
        cols.append(m[:, :, None, :])
    return jnp.concatenate(cols, axis=2)


def _bnrelu(z, s):
    return jnp.maximum(((z + s[0:1]) - s[1:2]) / s[2:3] * s[3:4] + s[4:5],
                       0.0)


def _trunk_body(xp_ref, w1_ref, s1_ref, w2_ref, s2_ref, w3_ref, s3_ref,
                w4_ref, s4_ref, out_ref, p1_ref, p3_ref, p4_ref):
    @pl.when(pl.program_id(0) == 0)
    def _():
        p1_ref[...] = jnp.zeros_like(p1_ref)
        p3_ref[...] = jnp.zeros_like(p3_ref)
        p4_ref[...] = jnp.zeros_like(p4_ref)

    xp = xp_ref[...].reshape(T, 34, 34, 3)
    a1 = _conv3x3(xp, w1_ref[...], T, 32, 32, 3, 32)
    a1 = _bnrelu(a1, s1_ref)
    p1_ref[:, 1:33, 1:33, :] = a1.reshape(T, 32, 32, 32)
    a2 = _conv3x3(p1_ref[...], w2_ref[...], T, 32, 32, 32, 64)
    a2 = _bnrelu(a2, s2_ref)
    a2 = _pool2x2(a2.reshape(T, 32, 32, 64), T, 32, 32, 64)
    p3_ref[:, 1:17, 1:17, :] = a2
    a3 = _conv3x3(p3_ref[...], w3_ref[...], T, 16, 16, 64, 128)
    a3 = _bnrelu(a3, s3_ref)
    p4_ref[:, 1:17, 1:17, :] = a3.reshape(T, 16, 16, 128)
    a4 = _conv3x3(p4_ref[...], w4_ref[...], T, 16, 16, 128, 256)
    a4 = _bnrelu(a4, s4_ref)
    a4 = _pool2x2(a4.reshape(T, 16, 16, 256), T, 16, 16, 256)
    out_ref[...] = jnp.mean(a4.reshape(T, 64, 256), axis=1)[None]


def _moe_body(f_ref, gw1_ref, gb1_ref, gw2_ref, gb2_ref, cw_ref, cb_ref,
              fl_ref, sc_ref, dm_ref):
    f = f_ref[...]
    h = jnp.maximum(jnp.dot(f, gw1_ref[...],
                            preferred_element_type=jnp.float32)
                    + gb1_ref[...], 0.0)
    gl = jnp.dot(h, gw2_ref[...],
                 preferred_element_type=jnp.float32) + gb2_ref[...]
    logits = []
    score_cols = []
    for e in range(E):
        le = jnp.dot(f, cw_ref[e],
                     preferred_element_type=jnp.float32) + cb_ref[e]
        m = jnp.max(le, axis=1, keepdims=True)
        p = jnp.exp(le - m)
        probs = p / jnp.sum(p, axis=1, keepdims=True)
        ent = -jnp.sum(probs * jnp.log(jnp.clip(probs, 1e-12, None)),
                       axis=1, keepdims=True)
        logits.append(le)
        score_cols.append(gl[:, e:e + 1] * (-ent))
    sc = jnp.concatenate(score_cols, axis=1)
    sc_ref[...] = sc
    scT = sc.T
    ii = jax.lax.broadcasted_iota(jnp.int32, (B, B), 0)
    jj = jax.lax.broadcasted_iota(jnp.int32, (B, B), 1)
    ones_col = jnp.ones((B, 1), jnp.float32)
    d1_cols = []
    for e in range(E):
        si = sc[:, e:e + 1]
        sj = scT[e:e + 1, :]
        ahead = (sj > si) | ((sj == si) & (jj < ii))
        rank = jnp.dot(ahead.astype(jnp.float32), ones_col,
                       preferred_element_type=jnp.float32)
        d1_cols.append((rank < float(min(CAP, B))).astype(jnp.float32))
    d1 = jnp.concatenate(d1_cols, axis=1)
    masked = jnp.where(d1 > 0.5, sc, -1e9)
    lane = jax.lax.broadcasted_iota(jnp.int32, (B, E), 1)
    d_cols = []
    for e in range(E):
        me = masked[:, e:e + 1]
        ahead = (masked > me) | ((masked == me) & (lane < e))
        rank = jnp.sum(ahead.astype(jnp.float32), axis=1, keepdims=True)
        d_cols.append((rank < float(K)).astype(jnp.float32))
    df = jnp.concatenate(d_cols, axis=1)
    dm_ref[...] = df
    w = sc * df
    acc = jnp.zeros((B, NUM_CLASSES), jnp.float32)
    for e in range(E):
        acc = acc + w[:, e:e + 1] * logits[e]
    norm = jnp.clip(jnp.sum(df, axis=1, keepdims=True), 1.0, None)
    fl_ref[...] = acc / norm


def _trunk_call(x, params, bn_stats):
    f32 = jnp.float32
    xp = jnp.pad(jnp.transpose(x, (0, 2, 3, 1)),
                 ((0, 0), (1, 1), (1, 1), (0, 0)))
    xm = xp.reshape(B, 34, 34 * 3)

    def stats(cb, g, bb, m, v):
        return jnp.stack([cb, m, jnp.sqrt(v + 1e-5), g, bb], axis=0)

    w1 = jnp.transpose(params['c1w'], (2, 3, 1, 0))
    w2 = jnp.transpose(params['c2w'], (2, 3, 1, 0))
    w3 = jnp.transpose(params['c3w'], (2, 3, 1, 0))
    w4 = jnp.transpose(params['c4w'], (2, 3, 1, 0))
    s1 = stats(params['c1b'], params['g1'], params['b1'],
               bn_stats['m1'], bn_stats['v1'])
    s2 = stats(params['c2b'], params['g2'], params['b2'],
               bn_stats['m2'], bn_stats['v2'])
    s3 = stats(params['c3b'], params['g3'], params['b3'],
               bn_stats['m3'], bn_stats['v3'])
    s4 = stats(params['c4b'], params['g4'], params['b4'],
               bn_stats['m4'], bn_stats['v4'])

    full = lambda shape: pl.BlockSpec(shape, lambda *a: (0,) * len(shape))
    feats = pl.pallas_call(
        _trunk_body,
        grid=(B // T,),
        in_specs=[
            pl.BlockSpec((T, 34, 102), lambda i: (i, 0, 0)),
            full((3, 3, 3, 32)), full((5, 32)),
            full((3, 3, 32, 64)), full((5, 64)),
            full((3, 3, 64, 128)), full((5, 128)),
            full((3, 3, 128, 256)), full((5, 256)),
        ],
        out_specs=pl.BlockSpec((1, T, TCDIM), lambda i: (i, 0, 0)),
        out_shape=jax.ShapeDtypeStruct((B // T, T, TCDIM), f32),
        scratch_shapes=[
            pltpu.VMEM((T, 34, 34, 32), f32),
            pltpu.VMEM((T, 18, 18, 64), f32),
            pltpu.VMEM((T, 18, 18, 128), f32),
        ],
    )(xm, w1, s1, w2, s2, w3, s3, w4, s4)
    return feats.reshape(B, TCDIM)


def _moe_call(feats, params):
    f32 = jnp.float32
    full = lambda shape: pl.BlockSpec(shape, lambda *a: (0,) * len(shape))
    gw1T = params['gw1'].T
    gb1 = params['gb1'][None, :]
    gw2T = params['gw2'].T
    gb2 = params['gb2'][None, :]
    cwT = jnp.transpose(params['cls_w'], (0, 2, 1))
    cb = params['cls_b'][:, None, :]

    fl, sc, dm = pl.pallas_call(
        _moe_body,
        in_specs=[
            full((B, TCDIM)),
            full((TCDIM, TCDIM // 2)), full((1, TCDIM // 2)),
            full((TCDIM // 2, E)), full((1, E)),
            full((E, TCDIM, NUM_CLASSES)), full((E, 1, NUM_CLASSES)),
        ],
        out_specs=[full((B, NUM_CLASSES)), full((B, E)), full((B, E))],
        out_shape=[
            jax.ShapeDtypeStruct((B, NUM_CLASSES), f32),
            jax.ShapeDtypeStruct((B, E), f32),
            jax.ShapeDtypeStruct((B, E), f32),
        ],
    )(feats, gw1T, gb1, gw2T, gb2, cwT, cb)
    return (fl, sc, dm)


def kernel(x, params, bn_stats):
    feats = _trunk_call(x, params, bn_stats)
    fl, sc, dm = _moe_call(feats, params)
    return (fl, sc, dm > 0.5)


# speedup vs baseline: 1.0073x; 1.0063x over previous
import jax
import jax.numpy as jnp
from jax.experimental import pallas as pl
from jax.experimental.pallas import tpu as pltpu

B = 256
HW = 32
E = 16
K = 2
CAP = 128
NUM_CLASSES = 10
TCDIM = 256

T = 8


def _conv3x3(xp, w, T_, H, W, Cin, Cout, splits=None):
    rows = T_ * H * W
    xs = [xp[:, dy:dy + H, dx:dx + W, :].reshape(rows, Cin)
          for dy in range(3) for dx in range(3)]
    lhs = jnp.concatenate(xs, axis=1)
    wf = w.reshape(9 * Cin, Cout)
    if splits is None:
        return jnp.dot(lhs, wf, preferred_element_type=jnp.float32)
    acc = jnp.zeros((rows, Cout), jnp.float32)
    k0 = 0
    for kl in splits:
        acc = acc + jnp.dot(lhs[:, k0:k0 + kl], wf[k0:k0 + kl],
                            preferred_element_type=jnp.float32)
        k0 += kl
    return acc


def _pool2x2(x, T_, H, W, C):
    xh = x.reshape(T_, H // 2, 2, W, C)
    xh = jnp.maximum(xh[:, :, 0], xh[:, :, 1])
    cols = []
    for w2 in range(W // 2):
        m = jnp.maximum(xh[:, :, 2 * w2, :], xh[:, :, 2 * w2 + 1, :])
        cols.append(m[:, :, None, :])
    return jnp.concatenate(cols, axis=2)


def _bnrelu(z, s):
    return jnp.maximum(((z + s[0:1]) - s[1:2]) / s[2:3] * s[3:4] + s[4:5],
                       0.0)


def _trunk_body(xp_ref, w1_ref, s1_ref, w2_ref, s2_ref, w3_ref, s3_ref,
                w4_ref, s4_ref, out_ref, p1_ref, p3_ref, p4_ref):
    @pl.when(pl.program_id(0) == 0)
    def _():
        p1_ref[...] = jnp.zeros_like(p1_ref)
        p3_ref[...] = jnp.zeros_like(p3_ref)
        p4_ref[...] = jnp.zeros_like(p4_ref)

    C = 2
    Ts = T // C
    for s in range(C):
        sl = slice(s * Ts, (s + 1) * Ts)
        xp = xp_ref[sl].reshape(Ts, 34, 34, 3)
        a1 = _conv3x3(xp, w1_ref[...], Ts, 32, 32, 3, 32)
        a1 = _bnrelu(a1, s1_ref)
        p1_ref[sl, 1:33, 1:33, :] = a1.reshape(Ts, 32, 32, 32)
        a2 = _conv3x3(p1_ref[sl], w2_ref[...], Ts, 32, 32, 32, 64)
        a2 = _bnrelu(a2, s2_ref)
        a2 = _pool2x2(a2.reshape(Ts, 32, 32, 64), Ts, 32, 32, 64)
        p3_ref[sl, 1:17, 1:17, :] = a2
        a3 = _conv3x3(p3_ref[sl], w3_ref[...], Ts, 16, 16, 64, 128)
        a3 = _bnrelu(a3, s3_ref)
        p4_ref[sl, 1:17, 1:17, :] = a3.reshape(Ts, 16, 16, 128)
        a4 = _conv3x3(p4_ref[sl], w4_ref[...], Ts, 16, 16, 128, 256)
        a4 = _bnrelu(a4, s4_ref)
        a4 = _pool2x2(a4.reshape(Ts, 16, 16, 256), Ts, 16, 16, 256)
        out_ref[0, sl] = jnp.mean(a4.reshape(Ts, 64, 256), axis=1)


def _moe_body(f_ref, gw1_ref, gb1_ref, gw2_ref, gb2_ref, cw_ref, cb_ref,
              fl_ref, sc_ref, dm_ref):
    f = f_ref[...]
    h = jnp.maximum(jnp.dot(f, gw1_ref[...],
                            preferred_element_type=jnp.float32)
                    + gb1_ref[...], 0.0)
    gl = jnp.dot(h, gw2_ref[...],
                 preferred_element_type=jnp.float32) + gb2_ref[...]
    logits = []
    score_cols = []
    for e in range(E):
        le = jnp.dot(f, cw_ref[e],
                     preferred_element_type=jnp.float32) + cb_ref[e]
        m = jnp.max(le, axis=1, keepdims=True)
        p = jnp.exp(le - m)
        probs = p / jnp.sum(p, axis=1, keepdims=True)
        ent = -jnp.sum(probs * jnp.log(jnp.clip(probs, 1e-12, None)),
                       axis=1, keepdims=True)
        logits.append(le)
        score_cols.append(gl[:, e:e + 1] * (-ent))
    sc = jnp.concatenate(score_cols, axis=1)
    sc_ref[...] = sc
    scT = sc.T
    ii = jax.lax.broadcasted_iota(jnp.int32, (B, B), 0)
    jj = jax.lax.broadcasted_iota(jnp.int32, (B, B), 1)
    ones_col = jnp.ones((B, 1), jnp.float32)
    d1_cols = []
    for e in range(E):
        si = sc[:, e:e + 1]
        sj = scT[e:e + 1, :]
        ahead = (sj > si) | ((sj == si) & (jj < ii))
        rank = jnp.dot(ahead.astype(jnp.float32), ones_col,
                       preferred_element_type=jnp.float32)
        d1_cols.append((rank < float(min(CAP, B))).astype(jnp.float32))
    d1 = jnp.concatenate(d1_cols, axis=1)
    masked = jnp.where(d1 > 0.5, sc, -1e9)
    lane = jax.lax.broadcasted_iota(jnp.int32, (B, E), 1)
    d_cols = []
    for e in range(E):
        me = masked[:, e:e + 1]
        ahead = (masked > me) | ((masked == me) & (lane < e))
        rank = jnp.sum(ahead.astype(jnp.float32), axis=1, keepdims=True)
        d_cols.append((rank < float(K)).astype(jnp.float32))
    df = jnp.concatenate(d_cols, axis=1)
    dm_ref[...] = df
    w = sc * df
    acc = jnp.zeros((B, NUM_CLASSES), jnp.float32)
    for e in range(E):
        acc = acc + w[:, e:e + 1] * logits[e]
    norm = jnp.clip(jnp.sum(df, axis=1, keepdims=True), 1.0, None)
    fl_ref[...] = acc / norm


def _trunk_call(x, params, bn_stats):
    f32 = jnp.float32
    xp = jnp.pad(jnp.transpose(x, (0, 2, 3, 1)),
                 ((0, 0), (1, 1), (1, 1), (0, 0)))
    xm = xp.reshape(B, 34, 34 * 3)

    def stats(cb, g, bb, m, v):
        return jnp.stack([cb, m, jnp.sqrt(v + 1e-5), g, bb], axis=0)

    w1 = jnp.transpose(params['c1w'], (2, 3, 1, 0))
    w2 = jnp.transpose(params['c2w'], (2, 3, 1, 0))
    w3 = jnp.transpose(params['c3w'], (2, 3, 1, 0))
    w4 = jnp.transpose(params['c4w'], (2, 3, 1, 0))
    s1 = stats(params['c1b'], params['g1'], params['b1'],
               bn_stats['m1'], bn_stats['v1'])
    s2 = stats(params['c2b'], params['g2'], params['b2'],
               bn_stats['m2'], bn_stats['v2'])
    s3 = stats(params['c3b'], params['g3'], params['b3'],
               bn_stats['m3'], bn_stats['v3'])
    s4 = stats(params['c4b'], params['g4'], params['b4'],
               bn_stats['m4'], bn_stats['v4'])

    full = lambda shape: pl.BlockSpec(shape, lambda *a: (0,) * len(shape))
    feats = pl.pallas_call(
        _trunk_body,
        grid=(B // T,),
        in_specs=[
            pl.BlockSpec((T, 34, 102), lambda i: (i, 0, 0)),
            full((3, 3, 3, 32)), full((5, 32)),
            full((3, 3, 32, 64)), full((5, 64)),
            full((3, 3, 64, 128)), full((5, 128)),
            full((3, 3, 128, 256)), full((5, 256)),
        ],
        out_specs=pl.BlockSpec((1, T, TCDIM), lambda i: (i, 0, 0)),
        out_shape=jax.ShapeDtypeStruct((B // T, T, TCDIM), f32),
        scratch_shapes=[
            pltpu.VMEM((T, 34, 34, 32), f32),
            pltpu.VMEM((T, 18, 18, 64), f32),
            pltpu.VMEM((T, 18, 18, 128), f32),
        ],
    )(xm, w1, s1, w2, s2, w3, s3, w4, s4)
    return feats.reshape(B, TCDIM)


def _moe_call(feats, params):
    f32 = jnp.float32
    full = lambda shape: pl.BlockSpec(shape, lambda *a: (0,) * len(shape))
    gw1T = params['gw1'].T
    gb1 = params['gb1'][None, :]
    gw2T = params['gw2'].T
    gb2 = params['gb2'][None, :]
    cwT = jnp.transpose(params['cls_w'], (0, 2, 1))
    cb = params['cls_b'][:, None, :]

    fl, sc, dm = pl.pallas_call(
        _moe_body,
        in_specs=[
            full((B, TCDIM)),
            full((TCDIM, TCDIM // 2)), full((1, TCDIM // 2)),
            full((TCDIM // 2, E)), full((1, E)),
            full((E, TCDIM, NUM_CLASSES)), full((E, 1, NUM_CLASSES)),
        ],
        out_specs=[full((B, NUM_CLASSES)), full((B, E)), full((B, E))],
        out_shape=[
            jax.ShapeDtypeStruct((B, NUM_CLASSES), f32),
            jax.ShapeDtypeStruct((B, E), f32),
            jax.ShapeDtypeStruct((B, E), f32),
        ],
    )(feats, gw1T, gb1, gw2T, gb2, cwT, cb)
    return (fl, sc, dm)


def kernel(x, params, bn_stats):
    feats = _trunk_call(x, params, bn_stats)
    fl, sc, dm = _moe_call(feats, params)
    return (fl, sc, dm > 0.5)


# speedup vs baseline: 1.0162x; 1.0088x over previous
import jax
import jax.numpy as jnp
from jax.experimental import pallas as pl
from jax.experimental.pallas import tpu as pltpu

B = 256
HW = 32
E = 16
K = 2
CAP = 128
NUM_CLASSES = 10
TCDIM = 256

T = 8


def _conv3x3(xp, w, T_, H, W, Cin, Cout, splits=None):
    rows = T_ * H * W
    xs = [xp[:, dy:dy + H, dx:dx + W, :].reshape(rows, Cin)
          for dy in range(3) for dx in range(3)]
    lhs = jnp.concatenate(xs, axis=1)
    wf = w.reshape(9 * Cin, Cout)
    if splits is None:
        return jnp.dot(lhs, wf, preferred_element_type=jnp.float32)
    acc = jnp.zeros((rows, Cout), jnp.float32)
    k0 = 0
    for kl in splits:
        acc = acc + jnp.dot(lhs[:, k0:k0 + kl], wf[k0:k0 + kl],
                            preferred_element_type=jnp.float32)
        k0 += kl
    return acc


def _conv3x3_ref(ref, sl, w, T_, H, W, Cin, Cout):
    rows = T_ * H * W
    xs = [ref[sl, dy:dy + H, dx:dx + W, :].reshape(rows, Cin)
          for dy in range(3) for dx in range(3)]
    lhs = jnp.concatenate(xs, axis=1)
    return jnp.dot(lhs, w.reshape(9 * Cin, Cout),
                   preferred_element_type=jnp.float32)


def _pool2x2(x, T_, H, W, C):
    xh = x.reshape(T_, H // 2, 2, W, C)
    xh = jnp.maximum(xh[:, :, 0], xh[:, :, 1])
    cols = []
    for w2 in range(W // 2):
        m = jnp.maximum(xh[:, :, 2 * w2, :], xh[:, :, 2 * w2 + 1, :])
        cols.append(m[:, :, None, :])
    return jnp.concatenate(cols, axis=2)


def _bnrelu(z, s):
    return jnp.maximum(((z + s[0:1]) - s[1:2]) / s[2:3] * s[3:4] + s[4:5],
                       0.0)


def _trunk_body(xp_ref, w1_ref, s1_ref, w2_ref, s2_ref, w3_ref, s3_ref,
                w4_ref, s4_ref, out_ref, p1_ref, p3_ref, p4_ref):
    @pl.when(pl.program_id(0) == 0)
    def _():
        p1_ref[...] = jnp.zeros_like(p1_ref)
        p3_ref[...] = jnp.zeros_like(p3_ref)
        p4_ref[...] = jnp.zeros_like(p4_ref)

    C = 2
    Ts = T // C
    for s in range(C):
        sl = slice(s * Ts, (s + 1) * Ts)
        xp = xp_ref[sl].reshape(Ts, 34, 34, 3)
        a1 = _conv3x3(xp, w1_ref[...], Ts, 32, 32, 3, 32)
        a1 = _bnrelu(a1, s1_ref)
        p1_ref[sl, 1:33, 1:33, :] = a1.reshape(Ts, 32, 32, 32)
        a2 = _conv3x3_ref(p1_ref, sl, w2_ref[...], Ts, 32, 32, 32, 64)
        a2 = _bnrelu(a2, s2_ref)
        a2 = _pool2x2(a2.reshape(Ts, 32, 32, 64), Ts, 32, 32, 64)
        p3_ref[sl, 1:17, 1:17, :] = a2
        a3 = _conv3x3_ref(p3_ref, sl, w3_ref[...], Ts, 16, 16, 64, 128)
        a3 = _bnrelu(a3, s3_ref)
        p4_ref[sl, 1:17, 1:17, :] = a3.reshape(Ts, 16, 16, 128)
        a4 = _conv3x3_ref(p4_ref, sl, w4_ref[...], Ts, 16, 16, 128, 256)
        a4 = _bnrelu(a4, s4_ref)
        a4 = _pool2x2(a4.reshape(Ts, 16, 16, 256), Ts, 16, 16, 256)
        out_ref[0, sl] = jnp.mean(a4.reshape(Ts, 64, 256), axis=1)


def _moe_body(f_ref, gw1_ref, gb1_ref, gw2_ref, gb2_ref, cw_ref, cb_ref,
              fl_ref, sc_ref, dm_ref):
    f = f_ref[...]
    h = jnp.maximum(jnp.dot(f, gw1_ref[...],
                            preferred_element_type=jnp.float32)
                    + gb1_ref[...], 0.0)
    gl = jnp.dot(h, gw2_ref[...],
                 preferred_element_type=jnp.float32) + gb2_ref[...]
    logits = []
    score_cols = []
    for e in range(E):
        le = jnp.dot(f, cw_ref[e],
                     preferred_element_type=jnp.float32) + cb_ref[e]
        m = jnp.max(le, axis=1, keepdims=True)
        p = jnp.exp(le - m)
        probs = p / jnp.sum(p, axis=1, keepdims=True)
        ent = -jnp.sum(probs * jnp.log(jnp.clip(probs, 1e-12, None)),
                       axis=1, keepdims=True)
        logits.append(le)
        score_cols.append(gl[:, e:e + 1] * (-ent))
    sc = jnp.concatenate(score_cols, axis=1)
    sc_ref[...] = sc
    scT = sc.T
    ii = jax.lax.broadcasted_iota(jnp.int32, (B, B), 0)
    jj = jax.lax.broadcasted_iota(jnp.int32, (B, B), 1)
    ones_col = jnp.ones((B, 1), jnp.float32)
    d1_cols = []
    for e in range(E):
        si = sc[:, e:e + 1]
        sj = scT[e:e + 1, :]
        ahead = (sj > si) | ((sj == si) & (jj < ii))
        rank = jnp.dot(ahead.astype(jnp.float32), ones_col,
                       preferred_element_type=jnp.float32)
        d1_cols.append((rank < float(min(CAP, B))).astype(jnp.float32))
    d1 = jnp.concatenate(d1_cols, axis=1)
    masked = jnp.where(d1 > 0.5, sc, -1e9)
    lane = jax.lax.broadcasted_iota(jnp.int32, (B, E), 1)
    d_cols = []
    for e in range(E):
        me = masked[:, e:e + 1]
        ahead = (masked > me) | ((masked == me) & (lane < e))
        rank = jnp.sum(ahead.astype(jnp.float32), axis=1, keepdims=True)
        d_cols.append((rank < float(K)).astype(jnp.float32))
    df = jnp.concatenate(d_cols, axis=1)
    dm_ref[...] = df
    w = sc * df
    acc = jnp.zeros((B, NUM_CLASSES), jnp.float32)
    for e in range(E):
        acc = acc + w[:, e:e + 1] * logits[e]
    norm = jnp.clip(jnp.sum(df, axis=1, keepdims=True), 1.0, None)
    fl_ref[...] = acc / norm


def _trunk_call(x, params, bn_stats):
    f32 = jnp.float32
    xp = jnp.pad(jnp.transpose(x, (0, 2, 3, 1)),
                 ((0, 0), (1, 1), (1, 1), (0, 0)))
    xm = xp.reshape(B, 34, 34 * 3)

    def stats(cb, g, bb, m, v):
        return jnp.stack([cb, m, jnp.sqrt(v + 1e-5), g, bb], axis=0)

    w1 = jnp.transpose(params['c1w'], (2, 3, 1, 0))
    w2 = jnp.transpose(params['c2w'], (2, 3, 1, 0))
    w3 = jnp.transpose(params['c3w'], (2, 3, 1, 0))
    w4 = jnp.transpose(params['c4w'], (2, 3, 1, 0))
    s1 = stats(params['c1b'], params['g1'], params['b1'],
               bn_stats['m1'], bn_stats['v1'])
    s2 = stats(params['c2b'], params['g2'], params['b2'],
               bn_stats['m2'], bn_stats['v2'])
    s3 = stats(params['c3b'], params['g3'], params['b3'],
               bn_stats['m3'], bn_stats['v3'])
    s4 = stats(params['c4b'], params['g4'], params['b4'],
               bn_stats['m4'], bn_stats['v4'])

    full = lambda shape: pl.BlockSpec(shape, lambda *a: (0,) * len(shape))
    feats = pl.pallas_call(
        _trunk_body,
        grid=(B // T,),
        in_specs=[
            pl.BlockSpec((T, 34, 102), lambda i: (i, 0, 0)),
            full((3, 3, 3, 32)), full((5, 32)),
            full((3, 3, 32, 64)), full((5, 64)),
            full((3, 3, 64, 128)), full((5, 128)),
            full((3, 3, 128, 256)), full((5, 256)),
        ],
        out_specs=pl.BlockSpec((1, T, TCDIM), lambda i: (i, 0, 0)),
        out_shape=jax.ShapeDtypeStruct((B // T, T, TCDIM), f32),
        scratch_shapes=[
            pltpu.VMEM((T, 34, 34, 32), f32),
            pltpu.VMEM((T, 18, 18, 64), f32),
            pltpu.VMEM((T, 18, 18, 128), f32),
        ],
    )(xm, w1, s1, w2, s2, w3, s3, w4, s4)
    return feats.reshape(B, TCDIM)


def _moe_call(feats, params):
    f32 = jnp.float32
    full = lambda shape: pl.BlockSpec(shape, lambda *a: (0,) * len(shape))
    gw1T = params['gw1'].T
    gb1 = params['gb1'][None, :]
    gw2T = params['gw2'].T
    gb2 = params['gb2'][None, :]
    cwT = jnp.transpose(params['cls_w'], (0, 2, 1))
    cb = params['cls_b'][:, None, :]

    fl, sc, dm = pl.pallas_call(
        _moe_body,
        in_specs=[
            full((B, TCDIM)),
            full((TCDIM, TCDIM // 2)), full((1, TCDIM // 2)),
            full((TCDIM // 2, E)), full((1, E)),
            full((E, TCDIM, NUM_CLASSES)), full((E, 1, NUM_CLASSES)),
        ],
        out_specs=[full((B, NUM_CLASSES)), full((B, E)), full((B, E))],
        out_shape=[
            jax.ShapeDtypeStruct((B, NUM_CLASSES), f32),
            jax.ShapeDtypeStruct((B, E), f32),
            jax.ShapeDtypeStruct((B, E), f32),
        ],
    )(feats, gw1T, gb1, gw2T, gb2, cwT, cb)
    return (fl, sc, dm)


def kernel(x, params, bn_stats):
    feats = _trunk_call(x, params, bn_stats)
    fl, sc, dm = _moe_call(feats, params)
    return (fl, sc, dm > 0.5)
